# baseline (speedup 1.0000x reference)
import functools

import jax
import jax.numpy as jnp
from jax import lax
from jax.experimental import pallas as pl
from jax.experimental.pallas import tpu as pltpu
from jax.experimental.pallas import tpu_sc as plsc

G = 2
W = 10000
S = 50000
D = 128
E = 300000

BATCH = 128
E_PAD = 303104
SLICE32 = E_PAD // 32
NB32 = SLICE32 // BATCH
SLICE16 = E_PAD // 16
NB16 = SLICE16 // BATCH

WPAD = 10240
W_TRASH = 10200
DEGPAD = 51200
DEG_TRASH = 51072
CHUNK = 6250
NCHUNK = 8
NBSC = 4
CHPAD = 6272
CH_TRASH = 6250
PSH = 13
PMASK = 8191

CAP = 19072
BIG1D = 32 * NBSC * CAP
CNT1D = G * 32 * 128

_mesh = plsc.VectorSubcoreMesh(core_axis_name="c", subcore_axis_name="s")
_IOTA16 = None


def _lanes(ref, j):
    return ref[pl.ds(16 * j, 16)]


@functools.partial(
    pl.kernel,
    out_type=(
        jax.ShapeDtypeStruct((WPAD,), jnp.float32),
        jax.ShapeDtypeStruct((WPAD,), jnp.float32),
        jax.ShapeDtypeStruct((DEGPAD,), jnp.float32),
        jax.ShapeDtypeStruct((DEGPAD,), jnp.float32),
    ),
    mesh=_mesh,
    scratch_types=(
        pltpu.VMEM((BATCH,), jnp.int32),
        pltpu.VMEM((BATCH,), jnp.int32),
        pltpu.VMEM((BATCH,), jnp.float32),
        pltpu.VMEM_SHARED((DEGPAD,), jnp.float32),
    ),
)
def _deg_kernel(r0, c0, r1, c1, ones_in, zeros1d, dw0, dw1, ds0, ds1,
                ebuf, sidx, ones_v, deg_sh):
    cid = lax.axis_index("c")
    sid = lax.axis_index("s")
    pltpu.sync_copy(ones_in, ones_v)
    iota = jnp.arange(16, dtype=jnp.int32)

    def scan(src_ref):
        def body(k, carry):
            base = pl.multiple_of(sid * SLICE16 + k * BATCH, 128)
            pltpu.sync_copy(src_ref.at[pl.ds(base, BATCH)], ebuf)
            for j in range(8):
                e = _lanes(ebuf, j)
                m = (base + 16 * j + iota) < E
                sidx[pl.ds(16 * j, 16)] = jnp.where(m, e, DEG_TRASH)
            pltpu.sync_copy(ones_v, deg_sh.at[sidx], add=True)
            return carry
        lax.fori_loop(0, NB16, body, 0)

    for rref, cref, wout, sout in ((r0, c0, dw0, ds0), (r1, c1, dw1, ds1)):
        pltpu.sync_copy(zeros1d, deg_sh.at[pl.ds(3200 * sid, 3200)])
        plsc.subcore_barrier()

        @pl.when(cid == 0)
        def _():
            scan(rref)

        @pl.when(cid == 1)
        def _():
            scan(cref)

        plsc.subcore_barrier()

        @pl.when(cid == 0)
        def _():
            pltpu.sync_copy(deg_sh.at[pl.ds(640 * sid, 640)],
                            wout.at[pl.ds(640 * sid, 640)])

        @pl.when(cid == 1)
        def _():
            pltpu.sync_copy(deg_sh.at[pl.ds(3200 * sid, 3200)],
                            sout.at[pl.ds(3200 * sid, 3200)])

        plsc.subcore_barrier()


@functools.partial(
    pl.kernel,
    out_type=(
        jax.ShapeDtypeStruct((2, WPAD, D), jnp.float32),
        jax.ShapeDtypeStruct((2, WPAD, D), jnp.float32),
    ),
    mesh=_mesh,
    scratch_types=(
        pltpu.VMEM((BATCH,), jnp.int32),
        pltpu.VMEM((BATCH,), jnp.int32),
        pltpu.VMEM((2, BATCH), jnp.int32),
        pltpu.VMEM((2, BATCH), jnp.int32),
        pltpu.VMEM((2, BATCH, D), jnp.float32),
        pltpu.VMEM_SHARED((WPAD, D), jnp.float32),
        pltpu.SemaphoreType.DMA,
        pltpu.SemaphoreType.DMA,
    ),
    compiler_params=pltpu.CompilerParams(needs_layout_passes=False),
)
def _spmm_w_kernel(r0, c0, gs0, r1, c1, gs1, zeros2d, hw0, hw1,
                   rbuf, cbuf, gidx, sidx, gbuf, hw_sh, sem, sem2):
    cid = lax.axis_index("c")
    sid = lax.axis_index("s")
    wstart = (cid * 16 + sid) * SLICE32
    iota = jnp.arange(16, dtype=jnp.int32)

    for b, (rref, cref, gref, href) in enumerate(
            ((r0, c0, gs0, hw0), (r1, c1, gs1, hw1))):
        pltpu.sync_copy(zeros2d.at[pl.ds(0, 640)],
                        hw_sh.at[pl.ds(640 * sid, 640)])
        plsc.subcore_barrier()

        def wait_add(k):
            par = k & 1
            pltpu.make_async_copy(gbuf.at[par], hw_sh.at[sidx.at[par]],
                                  sem2).wait()

        def fire_add(k):
            par = k & 1
            pltpu.make_async_copy(gbuf.at[par], hw_sh.at[sidx.at[par]],
                                  sem2).start(add=True)

        def build_fire(k):
            par = k & 1
            base = pl.multiple_of(wstart + k * BATCH, 128)
            pltpu.sync_copy(rref.at[pl.ds(base, BATCH)], rbuf)
            pltpu.sync_copy(cref.at[pl.ds(base, BATCH)], cbuf)
            for j in range(8):
                r16 = _lanes(rbuf, j)
                c16 = _lanes(cbuf, j)
                m = (base + 16 * j + iota) < E
                gidx[par, pl.ds(16 * j, 16)] = jnp.where(m, c16, 0)
                sidx[par, pl.ds(16 * j, 16)] = jnp.where(m, r16, W_TRASH)
            pltpu.async_copy(gref.at[gidx.at[par]], gbuf.at[par], sem)

        def wait_gather(k):
            par = k & 1
            pltpu.make_async_copy(gref.at[gidx.at[par]], gbuf.at[par],
                                  sem).wait()

        def body(k, carry):
            @pl.when(k >= 2)
            def _():
                wait_add(k - 2)
            build_fire(k)

            @pl.when(k >= 1)
            def _():
                wait_gather(k - 1)
                fire_add(k - 1)
            return carry
        lax.fori_loop(0, NB32, body, 0)
        wait_gather(jnp.int32(NB32 - 1))
        fire_add(jnp.int32(NB32 - 1))
        wait_add(jnp.int32(NB32 - 2))
        wait_add(jnp.int32(NB32 - 1))

        plsc.subcore_barrier()
        pltpu.sync_copy(hw_sh.at[pl.ds(640 * sid, 640)],
                        href.at[cid, pl.ds(640 * sid, 640)])
        plsc.subcore_barrier()


@functools.partial(
    pl.kernel,
    out_type=(
        jax.ShapeDtypeStruct((BIG1D,), jnp.int32),
        jax.ShapeDtypeStruct((BIG1D,), jnp.int32),
        jax.ShapeDtypeStruct((CNT1D,), jnp.int32),
    ),
    mesh=_mesh,
    scratch_types=(
        pltpu.VMEM((BATCH,), jnp.int32),
        pltpu.VMEM((BATCH,), jnp.int32),
        pltpu.VMEM((CAP,), jnp.int32),
        pltpu.VMEM((CAP,), jnp.int32),
        pltpu.VMEM((CAP,), jnp.int32),
        pltpu.VMEM((CAP,), jnp.int32),
        pltpu.VMEM((BATCH,), jnp.int32),
    ),
    compiler_params=pltpu.CompilerParams(needs_layout_passes=False),
)
def _bin_kernel(r0, c0, r1, c1, bp0, bp1, cnts,
                rbuf, cbuf, cpa, cpb, cpc, cpd, cntv):
    cid = lax.axis_index("c")
    sid = lax.axis_index("s")
    wid = cid * 16 + sid
    iota = jnp.arange(16, dtype=jnp.int32)
    lo = cid * NBSC * CHUNK

    for b, (rref, cref, opack) in enumerate(
            ((r0, c0, bp0), (r1, c1, bp1))):
        z16 = jnp.zeros((16,), jnp.int32)

        def _body(k, ptrs):
            ps = list(ptrs)
            base = pl.multiple_of(sid * SLICE16 + k * BATCH, 128)
            pltpu.sync_copy(rref.at[pl.ds(base, BATCH)], rbuf)
            pltpu.sync_copy(cref.at[pl.ds(base, BATCH)], cbuf)
            for j in range(8):
                r16 = _lanes(rbuf, j)
                c16 = _lanes(cbuf, j)
                m = (base + 16 * j + iota) < E
                lc = c16 - lo
                qv = ((lc >= CHUNK).astype(jnp.int32)
                      + (lc >= 2 * CHUNK).astype(jnp.int32)
                      + (lc >= 3 * CHUNK).astype(jnp.int32))
                lcq = lc - qv * CHUNK
                packed = r16 * (PMASK + 1) + lcq
                for q in range(NBSC):
                    inq = m & (lc >= q * CHUNK) & (lc < (q + 1) * CHUNK)
                    key = jnp.where(inq, iota, 16 + iota)
                    _, sv = plsc.sort_key_val(key, packed)
                    plsc.store_scatter((cpa, cpb, cpc, cpd)[q],
                                       [ps[q] + iota], sv)
                    ps[q] = ps[q] + plsc.all_reduce_population_count(inq)
            return tuple(ps)
        ns = lax.fori_loop(0, NB16, _body, (z16,) * NBSC)

        for q in range(NBSC):
            bq = pl.multiple_of((wid * NBSC + q) * CAP, 128)
            pltpu.sync_copy((cpa, cpb, cpc, cpd)[q],
                            opack.at[pl.ds(bq, CAP)])
        for j in range(8):
            v = jnp.zeros((16,), jnp.int32)
            if j == 0:
                for q in range(NBSC):
                    v = v + jnp.where(iota == q, ns[q], 0)
            cntv[pl.ds(16 * j, 16)] = v
        cbase = pl.multiple_of((b * 32 + wid) * 128, 128)
        pltpu.sync_copy(cntv, cnts.at[pl.ds(cbase, BATCH)])


@functools.partial(
    pl.kernel,
    out_type=(
        jax.ShapeDtypeStruct((NCHUNK, CHPAD, D), jnp.float32),
        jax.ShapeDtypeStruct((NCHUNK, CHPAD, D), jnp.float32),
    ),
    mesh=_mesh,
    scratch_types=(
        pltpu.VMEM((BATCH,), jnp.int32),
        pltpu.VMEM((2, BATCH), jnp.int32),
        pltpu.VMEM((2, BATCH), jnp.int32),
        pltpu.VMEM((2, BATCH, D), jnp.float32),
        pltpu.VMEM((BATCH,), jnp.int32),
        pltpu.VMEM_SHARED((CHPAD, D), jnp.float32),
        pltpu.SemaphoreType.DMA,
        pltpu.SemaphoreType.DMA,
    ),
    compiler_params=pltpu.CompilerParams(needs_layout_passes=False),
)
def _spmm_s_kernel(bpk0, gw0, bpk1, gw1, cnts, zeros2d,
                   hs0, hs1, pbuf, gidx, sidx, gbuf, cntv, ch_sh, sem, sem2):
    cid = lax.axis_index("c")
    sid = lax.axis_index("s")
    wid = cid * 16 + sid
    iota = jnp.arange(16, dtype=jnp.int32)

    for b, (pref, gref, href) in enumerate(
            ((bpk0, gw0, hs0), (bpk1, gw1, hs1))):
        cbase = pl.multiple_of((b * 32 + wid) * 128, 128)
        pltpu.sync_copy(cnts.at[pl.ds(cbase, BATCH)], cntv)
        c16 = cntv[pl.ds(0, 16)]
        for q in range(NBSC):
            chunk = cid * NBSC + q
            nq = jnp.sum(jnp.where(iota == q, c16, 0))
            nq = jnp.minimum(jnp.maximum(nq, 0), SLICE16)
            bbase = pl.multiple_of((wid * NBSC + q) * CAP, 128)
            pltpu.sync_copy(zeros2d.at[pl.ds(0, 392)],
                            ch_sh.at[pl.ds(392 * sid, 392)])
            plsc.subcore_barrier()

            def build_fire(k):
                par = k & 1
                base = pl.multiple_of(bbase + k * BATCH, 128)
                pltpu.sync_copy(pref.at[pl.ds(base, BATCH)], pbuf)
                for j in range(8):
                    pk = _lanes(pbuf, j)
                    r16 = jnp.right_shift(pk, PSH)
                    lc16 = pk & PMASK
                    m = (k * BATCH + 16 * j + iota) < nq
                    gidx[par, pl.ds(16 * j, 16)] = jnp.where(m, r16, 0)
                    sidx[par, pl.ds(16 * j, 16)] = jnp.where(m, lc16,
                                                             CH_TRASH)
                pltpu.async_copy(gref.at[gidx.at[par]], gbuf.at[par], sem)

            def wait_gather(k):
                par = k & 1
                pltpu.make_async_copy(gref.at[gidx.at[par]], gbuf.at[par],
                                      sem).wait()

            def fire_add(k):
                par = k & 1
                pltpu.make_async_copy(gbuf.at[par], ch_sh.at[sidx.at[par]],
                                      sem2).start(add=True)

            def wait_add(k):
                par = k & 1
                pltpu.make_async_copy(gbuf.at[par], ch_sh.at[sidx.at[par]],
                                      sem2).wait()

            nbatch = (nq + BATCH - 1) // BATCH

            def body(k, carry):
                @pl.when(k >= 2)
                def _():
                    wait_add(k - 2)
                build_fire(k)

                @pl.when(k >= 1)
                def _():
                    wait_gather(k - 1)
                    fire_add(k - 1)
                return carry
            lax.fori_loop(0, nbatch, body, 0)

            @pl.when(nbatch > 0)
            def _():
                wait_gather(nbatch - 1)
                fire_add(nbatch - 1)
                wait_add(nbatch - 1)

            @pl.when(nbatch > 1)
            def _():
                wait_add(nbatch - 2)

            plsc.subcore_barrier()

            @pl.when(sid < 15)
            def _():
                pltpu.sync_copy(ch_sh.at[pl.ds(392 * sid, 392)],
                                href.at[chunk, pl.ds(392 * sid, 392)])

            @pl.when(sid == 15)
            def _():
                pltpu.sync_copy(ch_sh.at[pl.ds(392 * 15, 376)],
                                href.at[chunk, pl.ds(392 * 15, 376)])

            plsc.subcore_barrier()


def _scale_body(deg_ref, f_ref, o_ref):
    w = 1.0 / (jnp.sqrt(deg_ref[...]) + 1e-8)
    o_ref[...] = f_ref[...] * w


def _scale(deg3, feats, rows, nb):
    blk = rows // nb
    return pl.pallas_call(
        _scale_body,
        grid=(G, nb),
        in_specs=[
            pl.BlockSpec((1, blk, 1), lambda b, i: (b, i, 0)),
            pl.BlockSpec((1, blk, D), lambda b, i: (b, i, 0)),
        ],
        out_specs=pl.BlockSpec((1, blk, D), lambda b, i: (b, i, 0)),
        out_shape=jax.ShapeDtypeStruct((G, rows, D), jnp.float32),
    )(deg3, feats)


def _post_body(deg_ref, hp_ref, acc_ref, acc_out, *rest, nparts, denom,
               want_g):
    w = 1.0 / (jnp.sqrt(deg_ref[...]) + 1e-8)
    h = hp_ref[:, 0]
    for p in range(1, nparts):
        h = h + hp_ref[:, p]
    f = (w * h) * (1.0 / denom)
    nrm = jnp.sqrt(jnp.sum(f * f, axis=2, keepdims=True))
    acc_out[...] = acc_ref[...] + f / jnp.maximum(nrm, 1e-12)
    if want_g:
        rest[0][...] = w * f


def _post(deg3, hparts, acc, rows, nparts, denom, want_g, nb):
    blk = rows // nb
    out_shape = [jax.ShapeDtypeStruct((G, rows, D), jnp.float32)]
    out_specs = [pl.BlockSpec((1, blk, D), lambda b, i: (b, i, 0))]
    if want_g:
        out_shape.append(jax.ShapeDtypeStruct((G, rows, D), jnp.float32))
        out_specs.append(pl.BlockSpec((1, blk, D), lambda b, i: (b, i, 0)))
    return pl.pallas_call(
        functools.partial(_post_body, nparts=nparts, denom=denom,
                          want_g=want_g),
        grid=(G, nb),
        in_specs=[
            pl.BlockSpec((1, blk, 1), lambda b, i: (b, i, 0)),
            pl.BlockSpec((1, nparts, blk, D), lambda b, i: (b, 0, i, 0)),
            pl.BlockSpec((1, blk, D), lambda b, i: (b, i, 0)),
        ],
        out_specs=out_specs,
        out_shape=out_shape,
    )(deg3, hparts, acc)


def kernel(edge_rows_0, edge_cols_0, edge_rows_1, edge_cols_1,
           warehouse_features, site_features):
    pad = lambda a: jnp.pad(a, (0, E_PAD - E))
    r0, c0 = pad(edge_rows_0), pad(edge_cols_0)
    r1, c1 = pad(edge_rows_1), pad(edge_cols_1)
    zeros2d = jnp.zeros((1024, D), jnp.float32)
    zeros1d = jnp.zeros((3200,), jnp.float32)
    ones128 = jnp.ones((BATCH,), jnp.float32)

    dw0, dw1, ds0, ds1 = _deg_kernel(r0, c0, r1, c1, ones128, zeros1d)
    degw3 = jnp.stack([dw0, dw1])[:, :, None]
    degs3 = jnp.stack([ds0, ds1])[:, :, None]

    gw = _scale(degw3, warehouse_features, W, 10)
    gs = _scale(degs3, site_features, S, 50)
    accw, accs = warehouse_features, site_features

    bp0, bp1, cnts = _bin_kernel(r0, c0, r1, c1)

    for i in range(2):
        hw0, hw1 = _spmm_w_kernel(r0, c0, gs[0], r1, c1, gs[1], zeros2d)
        hs0, hs1 = _spmm_s_kernel(bp0, gw[0], bp1, gw[1], cnts, zeros2d)
        hw = jnp.stack([hw0, hw1])[:, :, :W]
        hs = jnp.stack([hs0, hs1])[:, :, :CHUNK].reshape(G, 1, S, D)
        want_g = i == 0
        if want_g:
            accw, gw = _post(degw3, hw, accw, W, 2, i + 2, True, 10)
            accs, gs = _post(degs3, hs, accs, S, 1, i + 2, True, 50)
        else:
            accw, = _post(degw3, hw, accw, W, 2, i + 2, False, 10)
            accs, = _post(degs3, hs, accs, S, 1, i + 2, False, 50)
    return accw, accs


# speedup vs baseline: 1.0915x; 1.0915x over previous
import functools

import jax
import jax.numpy as jnp
from jax import lax
from jax.experimental import pallas as pl
from jax.experimental.pallas import tpu as pltpu
from jax.experimental.pallas import tpu_sc as plsc

G = 2
W = 10000
S = 50000
D = 128
E = 300000

BATCH = 128
E_PAD = 303104
SLICE32 = E_PAD // 32
NB32 = SLICE32 // BATCH
SLICE16 = E_PAD // 16
NB16 = SLICE16 // BATCH

WPAD = 10240
W_TRASH = 10200
DEGPAD = 51200
DEG_TRASH = 51072
CHUNK = 6250
NCHUNK = 8
NBSC = 4
CHPAD = 6272
CH_TRASH = 6250
PSH = 13
PMASK = 8191

CAP = 19072
BIG1D = 32 * NBSC * CAP
CNT1D = G * 32 * 128

_mesh = plsc.VectorSubcoreMesh(core_axis_name="c", subcore_axis_name="s")
_IOTA16 = None


def _lanes(ref, j):
    return ref[pl.ds(16 * j, 16)]


@functools.partial(
    pl.kernel,
    out_type=(
        jax.ShapeDtypeStruct((WPAD,), jnp.float32),
        jax.ShapeDtypeStruct((WPAD,), jnp.float32),
        jax.ShapeDtypeStruct((DEGPAD,), jnp.float32),
        jax.ShapeDtypeStruct((DEGPAD,), jnp.float32),
    ),
    mesh=_mesh,
    scratch_types=(
        pltpu.VMEM((BATCH,), jnp.int32),
        pltpu.VMEM((BATCH,), jnp.int32),
        pltpu.VMEM((BATCH,), jnp.float32),
        pltpu.VMEM_SHARED((DEGPAD,), jnp.float32),
    ),
)
def _deg_kernel(r0, c0, r1, c1, ones_in, zeros1d, dw0, dw1, ds0, ds1,
                ebuf, sidx, ones_v, deg_sh):
    cid = lax.axis_index("c")
    sid = lax.axis_index("s")
    pltpu.sync_copy(ones_in, ones_v)
    iota = jnp.arange(16, dtype=jnp.int32)

    def scan(src_ref):
        def body(k, carry):
            base = pl.multiple_of(sid * SLICE16 + k * BATCH, 128)
            pltpu.sync_copy(src_ref.at[pl.ds(base, BATCH)], ebuf)
            for j in range(8):
                e = _lanes(ebuf, j)
                m = (base + 16 * j + iota) < E
                sidx[pl.ds(16 * j, 16)] = jnp.where(m, e, DEG_TRASH)
            pltpu.sync_copy(ones_v, deg_sh.at[sidx], add=True)
            return carry
        lax.fori_loop(0, NB16, body, 0)

    for rref, cref, wout, sout in ((r0, c0, dw0, ds0), (r1, c1, dw1, ds1)):
        pltpu.sync_copy(zeros1d, deg_sh.at[pl.ds(3200 * sid, 3200)])
        plsc.subcore_barrier()

        @pl.when(cid == 0)
        def _():
            scan(rref)

        @pl.when(cid == 1)
        def _():
            scan(cref)

        plsc.subcore_barrier()

        @pl.when(cid == 0)
        def _():
            pltpu.sync_copy(deg_sh.at[pl.ds(640 * sid, 640)],
                            wout.at[pl.ds(640 * sid, 640)])

        @pl.when(cid == 1)
        def _():
            pltpu.sync_copy(deg_sh.at[pl.ds(3200 * sid, 3200)],
                            sout.at[pl.ds(3200 * sid, 3200)])

        plsc.subcore_barrier()


@functools.partial(
    pl.kernel,
    out_type=(
        jax.ShapeDtypeStruct((2, WPAD, D), jnp.float32),
        jax.ShapeDtypeStruct((2, WPAD, D), jnp.float32),
    ),
    mesh=_mesh,
    scratch_types=(
        pltpu.VMEM((4736,), jnp.int32),
        pltpu.VMEM((4736,), jnp.int32),
        pltpu.VMEM((2, BATCH), jnp.int32),
        pltpu.VMEM((2, BATCH), jnp.int32),
        pltpu.VMEM((2, BATCH, D), jnp.float32),
        pltpu.VMEM_SHARED((WPAD, D), jnp.float32),
        pltpu.SemaphoreType.DMA,
        pltpu.SemaphoreType.DMA,
    ),
    compiler_params=pltpu.CompilerParams(needs_layout_passes=False),
)
def _spmm_w_kernel(r0, c0, gs0, r1, c1, gs1, zeros2d, hw0, hw1,
                   rsl, csl, gidx, sidx, gbuf, hw_sh, sem, sem2):
    cid = lax.axis_index("c")
    sid = lax.axis_index("s")
    wstart = (cid * 16 + sid) * SLICE32
    iota = jnp.arange(16, dtype=jnp.int32)

    for b, (rref, cref, gref, href) in enumerate(
            ((r0, c0, gs0, hw0), (r1, c1, gs1, hw1))):
        pltpu.sync_copy(zeros2d.at[pl.ds(0, 640)],
                        hw_sh.at[pl.ds(640 * sid, 640)])
        plsc.subcore_barrier()

        def wait_add(k):
            par = k & 1
            pltpu.make_async_copy(gbuf.at[par], hw_sh.at[sidx.at[par]],
                                  sem2).wait()

        def fire_add(k):
            par = k & 1
            pltpu.make_async_copy(gbuf.at[par], hw_sh.at[sidx.at[par]],
                                  sem2).start(add=True)

        def build_fire(k, hb):
            par = k & 1
            for j in range(8):
                off = k * BATCH + 16 * j
                r16 = rsl[pl.ds(off, 16)]
                c16 = csl[pl.ds(off, 16)]
                m = (hb + off + iota) < E
                gidx[par, pl.ds(16 * j, 16)] = jnp.where(m, c16, 0)
                sidx[par, pl.ds(16 * j, 16)] = jnp.where(m, r16, W_TRASH)
            pltpu.async_copy(gref.at[gidx.at[par]], gbuf.at[par], sem)

        def wait_gather(k):
            par = k & 1
            pltpu.make_async_copy(gref.at[gidx.at[par]], gbuf.at[par],
                                  sem).wait()

        NBH = NB32 // 2
        for h in range(2):
            hb = pl.multiple_of(wstart + h * (NBH * BATCH), 128)
            pltpu.sync_copy(rref.at[pl.ds(hb, NBH * BATCH)], rsl)
            pltpu.sync_copy(cref.at[pl.ds(hb, NBH * BATCH)], csl)

            def body(k, carry, hb=hb):
                @pl.when(k >= 2)
                def _():
                    wait_add(k - 2)
                build_fire(k, hb)

                @pl.when(k >= 1)
                def _():
                    wait_gather(k - 1)
                    fire_add(k - 1)
                return carry
            lax.fori_loop(0, NBH, body, 0)
            wait_gather(jnp.int32(NBH - 1))
            fire_add(jnp.int32(NBH - 1))
            wait_add(jnp.int32(NBH - 2))
            wait_add(jnp.int32(NBH - 1))

        plsc.subcore_barrier()
        pltpu.sync_copy(hw_sh.at[pl.ds(640 * sid, 640)],
                        href.at[cid, pl.ds(640 * sid, 640)])
        plsc.subcore_barrier()


@functools.partial(
    pl.kernel,
    out_type=(
        jax.ShapeDtypeStruct((BIG1D,), jnp.int32),
        jax.ShapeDtypeStruct((BIG1D,), jnp.int32),
        jax.ShapeDtypeStruct((CNT1D,), jnp.int32),
    ),
    mesh=_mesh,
    scratch_types=(
        pltpu.VMEM((9472,), jnp.int32),
        pltpu.VMEM((9472,), jnp.int32),
        pltpu.VMEM((CAP,), jnp.int32),
        pltpu.VMEM((CAP,), jnp.int32),
        pltpu.VMEM((CAP,), jnp.int32),
        pltpu.VMEM((CAP,), jnp.int32),
        pltpu.VMEM((BATCH,), jnp.int32),
    ),
    compiler_params=pltpu.CompilerParams(needs_layout_passes=False),
)
def _bin_kernel(r0, c0, r1, c1, bp0, bp1, cnts,
                rsl, csl, cpa, cpb, cpc, cpd, cntv):
    cid = lax.axis_index("c")
    sid = lax.axis_index("s")
    wid = cid * 16 + sid
    iota = jnp.arange(16, dtype=jnp.int32)
    lo = cid * NBSC * CHUNK

    for b, (rref, cref, opack) in enumerate(
            ((r0, c0, bp0), (r1, c1, bp1))):
        z16 = jnp.zeros((16,), jnp.int32)

        def _body(k, ptrs, hb=0):
            ps = list(ptrs)
            for j in range(8):
                off = k * BATCH + 16 * j
                r16 = rsl[pl.ds(off, 16)]
                c16 = csl[pl.ds(off, 16)]
                m = (hb + off + iota) < E
                lc = c16 - lo
                qv = ((lc >= CHUNK).astype(jnp.int32)
                      + (lc >= 2 * CHUNK).astype(jnp.int32)
                      + (lc >= 3 * CHUNK).astype(jnp.int32))
                lcq = lc - qv * CHUNK
                packed = r16 * (PMASK + 1) + lcq
                for q in range(NBSC):
                    inq = m & (lc >= q * CHUNK) & (lc < (q + 1) * CHUNK)
                    key = jnp.where(inq, iota, 16 + iota)
                    _, sv = plsc.sort_key_val(key, packed)
                    plsc.store_scatter((cpa, cpb, cpc, cpd)[q],
                                       [ps[q] + iota], sv)
                    ps[q] = ps[q] + plsc.all_reduce_population_count(inq)
            return tuple(ps)

        NBH16 = NB16 // 2
        ns = (z16,) * NBSC
        for h in range(2):
            hb = pl.multiple_of(sid * SLICE16 + h * (NBH16 * BATCH), 128)
            pltpu.sync_copy(rref.at[pl.ds(hb, NBH16 * BATCH)], rsl)
            pltpu.sync_copy(cref.at[pl.ds(hb, NBH16 * BATCH)], csl)
            ns = lax.fori_loop(0, NBH16,
                               functools.partial(_body, hb=hb), ns)

        for q in range(NBSC):
            bq = pl.multiple_of((wid * NBSC + q) * CAP, 128)
            pltpu.sync_copy((cpa, cpb, cpc, cpd)[q],
                            opack.at[pl.ds(bq, CAP)])
        for j in range(8):
            v = jnp.zeros((16,), jnp.int32)
            if j == 0:
                for q in range(NBSC):
                    v = v + jnp.where(iota == q, ns[q], 0)
            cntv[pl.ds(16 * j, 16)] = v
        cbase = pl.multiple_of((b * 32 + wid) * 128, 128)
        pltpu.sync_copy(cntv, cnts.at[pl.ds(cbase, BATCH)])


@functools.partial(
    pl.kernel,
    out_type=(
        jax.ShapeDtypeStruct((NCHUNK, CHPAD, D), jnp.float32),
        jax.ShapeDtypeStruct((NCHUNK, CHPAD, D), jnp.float32),
    ),
    mesh=_mesh,
    scratch_types=(
        pltpu.VMEM((CAP,), jnp.int32),
        pltpu.VMEM((2, BATCH), jnp.int32),
        pltpu.VMEM((2, BATCH), jnp.int32),
        pltpu.VMEM((2, BATCH, D), jnp.float32),
        pltpu.VMEM((BATCH,), jnp.int32),
        pltpu.VMEM_SHARED((CHPAD, D), jnp.float32),
        pltpu.SemaphoreType.DMA,
        pltpu.SemaphoreType.DMA,
    ),
    compiler_params=pltpu.CompilerParams(needs_layout_passes=False),
)
def _spmm_s_kernel(bpk0, gw0, bpk1, gw1, cnts, zeros2d,
                   hs0, hs1, psl, gidx, sidx, gbuf, cntv, ch_sh, sem, sem2):
    cid = lax.axis_index("c")
    sid = lax.axis_index("s")
    wid = cid * 16 + sid
    iota = jnp.arange(16, dtype=jnp.int32)

    for b, (pref, gref, href) in enumerate(
            ((bpk0, gw0, hs0), (bpk1, gw1, hs1))):
        cbase = pl.multiple_of((b * 32 + wid) * 128, 128)
        pltpu.sync_copy(cnts.at[pl.ds(cbase, BATCH)], cntv)
        c16 = cntv[pl.ds(0, 16)]
        for q in range(NBSC):
            chunk = cid * NBSC + q
            nq = jnp.sum(jnp.where(iota == q, c16, 0))
            nq = jnp.minimum(jnp.maximum(nq, 0), SLICE16)
            bbase = pl.multiple_of((wid * NBSC + q) * CAP, 128)
            pltpu.sync_copy(zeros2d.at[pl.ds(0, 392)],
                            ch_sh.at[pl.ds(392 * sid, 392)])
            plsc.subcore_barrier()

            pltpu.sync_copy(pref.at[pl.ds(bbase, CAP)], psl)

            def build_fire(k):
                par = k & 1
                for j in range(8):
                    off = k * BATCH + 16 * j
                    pk = psl[pl.ds(off, 16)]
                    r16 = jnp.right_shift(pk, PSH)
                    lc16 = pk & PMASK
                    m = (off + iota) < nq
                    gidx[par, pl.ds(16 * j, 16)] = jnp.where(m, r16, 0)
                    sidx[par, pl.ds(16 * j, 16)] = jnp.where(m, lc16,
                                                             CH_TRASH)
                pltpu.async_copy(gref.at[gidx.at[par]], gbuf.at[par], sem)

            def wait_gather(k):
                par = k & 1
                pltpu.make_async_copy(gref.at[gidx.at[par]], gbuf.at[par],
                                      sem).wait()

            def fire_add(k):
                par = k & 1
                pltpu.make_async_copy(gbuf.at[par], ch_sh.at[sidx.at[par]],
                                      sem2).start(add=True)

            def wait_add(k):
                par = k & 1
                pltpu.make_async_copy(gbuf.at[par], ch_sh.at[sidx.at[par]],
                                      sem2).wait()

            nbatch = (nq + BATCH - 1) // BATCH

            def body(k, carry):
                @pl.when(k >= 2)
                def _():
                    wait_add(k - 2)
                build_fire(k)

                @pl.when(k >= 1)
                def _():
                    wait_gather(k - 1)
                    fire_add(k - 1)
                return carry
            lax.fori_loop(0, nbatch, body, 0)

            @pl.when(nbatch > 0)
            def _():
                wait_gather(nbatch - 1)
                fire_add(nbatch - 1)
                wait_add(nbatch - 1)

            @pl.when(nbatch > 1)
            def _():
                wait_add(nbatch - 2)

            plsc.subcore_barrier()

            @pl.when(sid < 15)
            def _():
                pltpu.sync_copy(ch_sh.at[pl.ds(392 * sid, 392)],
                                href.at[chunk, pl.ds(392 * sid, 392)])

            @pl.when(sid == 15)
            def _():
                pltpu.sync_copy(ch_sh.at[pl.ds(392 * 15, 376)],
                                href.at[chunk, pl.ds(392 * 15, 376)])

            plsc.subcore_barrier()


def _scale_body(deg_ref, f_ref, o_ref):
    w = 1.0 / (jnp.sqrt(deg_ref[...]) + 1e-8)
    o_ref[...] = f_ref[...] * w


def _scale(deg3, feats, rows, nb):
    blk = rows // nb
    return pl.pallas_call(
        _scale_body,
        grid=(G, nb),
        in_specs=[
            pl.BlockSpec((1, blk, 1), lambda b, i: (b, i, 0)),
            pl.BlockSpec((1, blk, D), lambda b, i: (b, i, 0)),
        ],
        out_specs=pl.BlockSpec((1, blk, D), lambda b, i: (b, i, 0)),
        out_shape=jax.ShapeDtypeStruct((G, rows, D), jnp.float32),
    )(deg3, feats)


def _post_body(deg_ref, hp_ref, acc_ref, acc_out, *rest, nparts, denom,
               want_g):
    w = 1.0 / (jnp.sqrt(deg_ref[...]) + 1e-8)
    h = hp_ref[:, 0]
    for p in range(1, nparts):
        h = h + hp_ref[:, p]
    f = (w * h) * (1.0 / denom)
    nrm = jnp.sqrt(jnp.sum(f * f, axis=2, keepdims=True))
    acc_out[...] = acc_ref[...] + f / jnp.maximum(nrm, 1e-12)
    if want_g:
        rest[0][...] = w * f


def _post(deg3, hparts, acc, rows, nparts, denom, want_g, nb):
    blk = rows // nb
    out_shape = [jax.ShapeDtypeStruct((G, rows, D), jnp.float32)]
    out_specs = [pl.BlockSpec((1, blk, D), lambda b, i: (b, i, 0))]
    if want_g:
        out_shape.append(jax.ShapeDtypeStruct((G, rows, D), jnp.float32))
        out_specs.append(pl.BlockSpec((1, blk, D), lambda b, i: (b, i, 0)))
    return pl.pallas_call(
        functools.partial(_post_body, nparts=nparts, denom=denom,
                          want_g=want_g),
        grid=(G, nb),
        in_specs=[
            pl.BlockSpec((1, blk, 1), lambda b, i: (b, i, 0)),
            pl.BlockSpec((1, nparts, blk, D), lambda b, i: (b, 0, i, 0)),
            pl.BlockSpec((1, blk, D), lambda b, i: (b, i, 0)),
        ],
        out_specs=out_specs,
        out_shape=out_shape,
    )(deg3, hparts, acc)


def kernel(edge_rows_0, edge_cols_0, edge_rows_1, edge_cols_1,
           warehouse_features, site_features):
    pad = lambda a: jnp.pad(a, (0, E_PAD - E))
    r0, c0 = pad(edge_rows_0), pad(edge_cols_0)
    r1, c1 = pad(edge_rows_1), pad(edge_cols_1)
    zeros2d = jnp.zeros((1024, D), jnp.float32)
    zeros1d = jnp.zeros((3200,), jnp.float32)
    ones128 = jnp.ones((BATCH,), jnp.float32)

    dw0, dw1, ds0, ds1 = _deg_kernel(r0, c0, r1, c1, ones128, zeros1d)
    degw3 = jnp.stack([dw0, dw1])[:, :, None]
    degs3 = jnp.stack([ds0, ds1])[:, :, None]

    gw = _scale(degw3, warehouse_features, W, 10)
    gs = _scale(degs3, site_features, S, 50)
    accw, accs = warehouse_features, site_features

    bp0, bp1, cnts = _bin_kernel(r0, c0, r1, c1)

    for i in range(2):
        hw0, hw1 = _spmm_w_kernel(r0, c0, gs[0], r1, c1, gs[1], zeros2d)
        hs0, hs1 = _spmm_s_kernel(bp0, gw[0], bp1, gw[1], cnts, zeros2d)
        hw = jnp.stack([hw0, hw1])[:, :, :W]
        hs = jnp.stack([hs0, hs1])[:, :, :CHUNK].reshape(G, 1, S, D)
        want_g = i == 0
        if want_g:
            accw, gw = _post(degw3, hw, accw, W, 2, i + 2, True, 10)
            accs, gs = _post(degs3, hs, accs, S, 1, i + 2, True, 50)
        else:
            accw, = _post(degw3, hw, accw, W, 2, i + 2, False, 10)
            accs, = _post(degs3, hs, accs, S, 1, i + 2, False, 50)
    return accw, accs


# speedup vs baseline: 1.1483x; 1.0521x over previous
import functools

import jax
import jax.numpy as jnp
from jax import lax
from jax.experimental import pallas as pl
from jax.experimental.pallas import tpu as pltpu
from jax.experimental.pallas import tpu_sc as plsc

G = 2
W = 10000
S = 50000
D = 128
E = 300000

BATCH = 128
E_PAD = 303104
SLICE32 = E_PAD // 32
NB32 = SLICE32 // BATCH
SLICE16 = E_PAD // 16
NB16 = SLICE16 // BATCH

WPAD = 10240
W_TRASH = 10200
DEGPAD = 51200
DEG_TRASH = 51072
CHUNK = 6250
NCHUNK = 8
NBSC = 4
CHPAD = 6272
CH_TRASH = 6250
PSH = 13
PMASK = 8191

CAP = 19072
BIG1D = 32 * NBSC * CAP
CNT1D = G * 32 * 128

_mesh = plsc.VectorSubcoreMesh(core_axis_name="c", subcore_axis_name="s")
_IOTA16 = None


def _lanes(ref, j):
    return ref[pl.ds(16 * j, 16)]


@functools.partial(
    pl.kernel,
    out_type=(
        jax.ShapeDtypeStruct((WPAD,), jnp.float32),
        jax.ShapeDtypeStruct((WPAD,), jnp.float32),
        jax.ShapeDtypeStruct((DEGPAD,), jnp.float32),
        jax.ShapeDtypeStruct((DEGPAD,), jnp.float32),
    ),
    mesh=_mesh,
    scratch_types=(
        pltpu.VMEM((BATCH,), jnp.int32),
        pltpu.VMEM((BATCH,), jnp.int32),
        pltpu.VMEM((BATCH,), jnp.float32),
        pltpu.VMEM_SHARED((DEGPAD,), jnp.float32),
    ),
)
def _deg_kernel(r0, c0, r1, c1, ones_in, zeros1d, dw0, dw1, ds0, ds1,
                ebuf, sidx, ones_v, deg_sh):
    cid = lax.axis_index("c")
    sid = lax.axis_index("s")
    pltpu.sync_copy(ones_in, ones_v)
    iota = jnp.arange(16, dtype=jnp.int32)

    def scan(src_ref):
        def body(k, carry):
            base = pl.multiple_of(sid * SLICE16 + k * BATCH, 128)
            pltpu.sync_copy(src_ref.at[pl.ds(base, BATCH)], ebuf)
            for j in range(8):
                e = _lanes(ebuf, j)
                m = (base + 16 * j + iota) < E
                sidx[pl.ds(16 * j, 16)] = jnp.where(m, e, DEG_TRASH)
            pltpu.sync_copy(ones_v, deg_sh.at[sidx], add=True)
            return carry
        lax.fori_loop(0, NB16, body, 0)

    for rref, cref, wout, sout in ((r0, c0, dw0, ds0), (r1, c1, dw1, ds1)):
        pltpu.sync_copy(zeros1d, deg_sh.at[pl.ds(3200 * sid, 3200)])
        plsc.subcore_barrier()

        @pl.when(cid == 0)
        def _():
            scan(rref)

        @pl.when(cid == 1)
        def _():
            scan(cref)

        plsc.subcore_barrier()

        @pl.when(cid == 0)
        def _():
            pltpu.sync_copy(deg_sh.at[pl.ds(640 * sid, 640)],
                            wout.at[pl.ds(640 * sid, 640)])

        @pl.when(cid == 1)
        def _():
            pltpu.sync_copy(deg_sh.at[pl.ds(3200 * sid, 3200)],
                            sout.at[pl.ds(3200 * sid, 3200)])

        plsc.subcore_barrier()


@functools.partial(
    pl.kernel,
    out_type=(
        jax.ShapeDtypeStruct((2, WPAD, D), jnp.float32),
        jax.ShapeDtypeStruct((2, WPAD, D), jnp.float32),
    ),
    mesh=_mesh,
    scratch_types=(
        pltpu.VMEM((4736,), jnp.int32),
        pltpu.VMEM((4736,), jnp.int32),
        pltpu.VMEM((2, BATCH), jnp.int32),
        pltpu.VMEM((2, BATCH), jnp.int32),
        pltpu.VMEM((2, BATCH, D), jnp.float32),
        pltpu.VMEM_SHARED((WPAD, D), jnp.float32),
        pltpu.SemaphoreType.DMA,
        pltpu.SemaphoreType.DMA,
    ),
    compiler_params=pltpu.CompilerParams(needs_layout_passes=False),
)
def _spmm_w_kernel(r0, c0, gs0, r1, c1, gs1, zeros2d, hw0, hw1,
                   rsl, csl, gidx, sidx, gbuf, hw_sh, sem, sem2):
    cid = lax.axis_index("c")
    sid = lax.axis_index("s")
    wstart = (cid * 16 + sid) * SLICE32
    iota = jnp.arange(16, dtype=jnp.int32)

    for b, (rref, cref, gref, href) in enumerate(
            ((r0, c0, gs0, hw0), (r1, c1, gs1, hw1))):
        pltpu.sync_copy(zeros2d.at[pl.ds(0, 640)],
                        hw_sh.at[pl.ds(640 * sid, 640)])
        plsc.subcore_barrier()

        def wait_add(k):
            par = k & 1
            pltpu.make_async_copy(gbuf.at[par], hw_sh.at[sidx.at[par]],
                                  sem2).wait()

        def fire_add(k):
            par = k & 1
            pltpu.make_async_copy(gbuf.at[par], hw_sh.at[sidx.at[par]],
                                  sem2).start(add=True)

        def build_fire(k, hb):
            par = k & 1
            for j in range(8):
                off = k * BATCH + 16 * j
                r16 = rsl[pl.ds(off, 16)]
                c16 = csl[pl.ds(off, 16)]
                m = (hb + off + iota) < E
                gidx[par, pl.ds(16 * j, 16)] = jnp.where(m, c16, 0)
                sidx[par, pl.ds(16 * j, 16)] = jnp.where(m, r16, W_TRASH)
            pltpu.async_copy(gref.at[gidx.at[par]], gbuf.at[par], sem)

        def wait_gather(k):
            par = k & 1
            pltpu.make_async_copy(gref.at[gidx.at[par]], gbuf.at[par],
                                  sem).wait()

        NBH = NB32 // 2
        for h in range(2):
            hb = pl.multiple_of(wstart + h * (NBH * BATCH), 128)
            pltpu.sync_copy(rref.at[pl.ds(hb, NBH * BATCH)], rsl)
            pltpu.sync_copy(cref.at[pl.ds(hb, NBH * BATCH)], csl)

            def body(k, carry, hb=hb):
                @pl.when(k >= 2)
                def _():
                    wait_add(k - 2)
                build_fire(k, hb)

                @pl.when(k >= 1)
                def _():
                    wait_gather(k - 1)
                    fire_add(k - 1)
                return carry
            lax.fori_loop(0, NBH, body, 0)
            wait_gather(jnp.int32(NBH - 1))
            fire_add(jnp.int32(NBH - 1))
            wait_add(jnp.int32(NBH - 2))
            wait_add(jnp.int32(NBH - 1))

        plsc.subcore_barrier()
        pltpu.sync_copy(hw_sh.at[pl.ds(640 * sid, 640)],
                        href.at[cid, pl.ds(640 * sid, 640)])
        plsc.subcore_barrier()


@functools.partial(
    pl.kernel,
    out_type=(
        jax.ShapeDtypeStruct((BIG1D,), jnp.int32),
        jax.ShapeDtypeStruct((BIG1D,), jnp.int32),
        jax.ShapeDtypeStruct((CNT1D,), jnp.int32),
        jax.ShapeDtypeStruct((WPAD,), jnp.float32),
        jax.ShapeDtypeStruct((WPAD,), jnp.float32),
        jax.ShapeDtypeStruct((DEGPAD,), jnp.float32),
        jax.ShapeDtypeStruct((DEGPAD,), jnp.float32),
    ),
    mesh=_mesh,
    scratch_types=(
        pltpu.VMEM((9472,), jnp.int32),
        pltpu.VMEM((9472,), jnp.int32),
        pltpu.VMEM((CAP,), jnp.int32),
        pltpu.VMEM((CAP,), jnp.int32),
        pltpu.VMEM((CAP,), jnp.int32),
        pltpu.VMEM((CAP,), jnp.int32),
        pltpu.VMEM((BATCH,), jnp.int32),
        pltpu.VMEM((BATCH,), jnp.float32),
        pltpu.VMEM((2, BATCH), jnp.int32),
        pltpu.VMEM_SHARED((DEGPAD,), jnp.float32),
        pltpu.SemaphoreType.DMA,
    ),
    compiler_params=pltpu.CompilerParams(needs_layout_passes=False),
)
def _bin_kernel(r0, c0, r1, c1, ones_in, zeros1d, bp0, bp1, cnts,
                dw0, dw1, ds0, ds1,
                rsl, csl, cpa, cpb, cpc, cpd, cntv, ones_v, didx, deg_sh,
                sem3):
    cid = lax.axis_index("c")
    sid = lax.axis_index("s")
    wid = cid * 16 + sid
    iota = jnp.arange(16, dtype=jnp.int32)
    lo = cid * NBSC * CHUNK
    pltpu.sync_copy(ones_in, ones_v)

    def wait_deg(k):
        par = k & 1
        pltpu.make_async_copy(ones_v, deg_sh.at[didx.at[par]], sem3).wait()

    for b, (rref, cref, opack, wout, sout) in enumerate(
            ((r0, c0, bp0, dw0, ds0), (r1, c1, bp1, dw1, ds1))):
        z16 = jnp.zeros((16,), jnp.int32)
        pltpu.sync_copy(zeros1d, deg_sh.at[pl.ds(3200 * sid, 3200)])
        plsc.subcore_barrier()

        def _body(k, ptrs, hb=0):
            ps = list(ptrs)

            @pl.when(k >= 2)
            def _():
                wait_deg(k - 2)
            par = k & 1
            for j in range(8):
                off = k * BATCH + 16 * j
                r16 = rsl[pl.ds(off, 16)]
                c16 = csl[pl.ds(off, 16)]
                m = (hb + off + iota) < E
                d16 = jnp.where(cid == 0, r16, c16)
                didx[par, pl.ds(16 * j, 16)] = jnp.where(m, d16, DEG_TRASH)
                lc = c16 - lo
                qv = ((lc >= CHUNK).astype(jnp.int32)
                      + (lc >= 2 * CHUNK).astype(jnp.int32)
                      + (lc >= 3 * CHUNK).astype(jnp.int32))
                lcq = lc - qv * CHUNK
                packed = r16 * (PMASK + 1) + lcq
                for q in range(NBSC):
                    inq = m & (lc >= q * CHUNK) & (lc < (q + 1) * CHUNK)
                    key = jnp.where(inq, iota, 16 + iota)
                    _, sv = plsc.sort_key_val(key, packed)
                    plsc.store_scatter((cpa, cpb, cpc, cpd)[q],
                                       [ps[q] + iota], sv)
                    ps[q] = ps[q] + plsc.all_reduce_population_count(inq)
            pltpu.make_async_copy(ones_v, deg_sh.at[didx.at[par]],
                                  sem3).start(add=True)
            return tuple(ps)

        NBH16 = NB16 // 2
        ns = (z16,) * NBSC
        for h in range(2):
            hb = pl.multiple_of(sid * SLICE16 + h * (NBH16 * BATCH), 128)
            pltpu.sync_copy(rref.at[pl.ds(hb, NBH16 * BATCH)], rsl)
            pltpu.sync_copy(cref.at[pl.ds(hb, NBH16 * BATCH)], csl)
            ns = lax.fori_loop(0, NBH16,
                               functools.partial(_body, hb=hb), ns)
            wait_deg(jnp.int32(NBH16 - 2))
            wait_deg(jnp.int32(NBH16 - 1))

        for q in range(NBSC):
            bq = pl.multiple_of((wid * NBSC + q) * CAP, 128)
            pltpu.sync_copy((cpa, cpb, cpc, cpd)[q],
                            opack.at[pl.ds(bq, CAP)])
        for j in range(8):
            v = jnp.zeros((16,), jnp.int32)
            if j == 0:
                for q in range(NBSC):
                    v = v + jnp.where(iota == q, ns[q], 0)
            cntv[pl.ds(16 * j, 16)] = v
        cbase = pl.multiple_of((b * 32 + wid) * 128, 128)
        pltpu.sync_copy(cntv, cnts.at[pl.ds(cbase, BATCH)])
        plsc.subcore_barrier()

        @pl.when(cid == 0)
        def _():
            pltpu.sync_copy(deg_sh.at[pl.ds(640 * sid, 640)],
                            wout.at[pl.ds(640 * sid, 640)])

        @pl.when(cid == 1)
        def _():
            pltpu.sync_copy(deg_sh.at[pl.ds(3200 * sid, 3200)],
                            sout.at[pl.ds(3200 * sid, 3200)])

        plsc.subcore_barrier()


@functools.partial(
    pl.kernel,
    out_type=(
        jax.ShapeDtypeStruct((NCHUNK, CHPAD, D), jnp.float32),
        jax.ShapeDtypeStruct((NCHUNK, CHPAD, D), jnp.float32),
    ),
    mesh=_mesh,
    scratch_types=(
        pltpu.VMEM((CAP,), jnp.int32),
        pltpu.VMEM((2, BATCH), jnp.int32),
        pltpu.VMEM((2, BATCH), jnp.int32),
        pltpu.VMEM((2, BATCH, D), jnp.float32),
        pltpu.VMEM((BATCH,), jnp.int32),
        pltpu.VMEM_SHARED((CHPAD, D), jnp.float32),
        pltpu.SemaphoreType.DMA,
        pltpu.SemaphoreType.DMA,
    ),
    compiler_params=pltpu.CompilerParams(needs_layout_passes=False),
)
def _spmm_s_kernel(bpk0, gw0, bpk1, gw1, cnts, zeros2d,
                   hs0, hs1, psl, gidx, sidx, gbuf, cntv, ch_sh, sem, sem2):
    cid = lax.axis_index("c")
    sid = lax.axis_index("s")
    wid = cid * 16 + sid
    iota = jnp.arange(16, dtype=jnp.int32)

    for b, (pref, gref, href) in enumerate(
            ((bpk0, gw0, hs0), (bpk1, gw1, hs1))):
        cbase = pl.multiple_of((b * 32 + wid) * 128, 128)
        pltpu.sync_copy(cnts.at[pl.ds(cbase, BATCH)], cntv)
        c16 = cntv[pl.ds(0, 16)]
        for q in range(NBSC):
            chunk = cid * NBSC + q
            nq = jnp.sum(jnp.where(iota == q, c16, 0))
            nq = jnp.minimum(jnp.maximum(nq, 0), SLICE16)
            bbase = pl.multiple_of((wid * NBSC + q) * CAP, 128)
            pltpu.sync_copy(zeros2d.at[pl.ds(0, 392)],
                            ch_sh.at[pl.ds(392 * sid, 392)])
            plsc.subcore_barrier()

            pltpu.sync_copy(pref.at[pl.ds(bbase, CAP)], psl)

            def build_fire(k):
                par = k & 1
                for j in range(8):
                    off = k * BATCH + 16 * j
                    pk = psl[pl.ds(off, 16)]
                    r16 = jnp.right_shift(pk, PSH)
                    lc16 = pk & PMASK
                    m = (off + iota) < nq
                    gidx[par, pl.ds(16 * j, 16)] = jnp.where(m, r16, 0)
                    sidx[par, pl.ds(16 * j, 16)] = jnp.where(m, lc16,
                                                             CH_TRASH)
                pltpu.async_copy(gref.at[gidx.at[par]], gbuf.at[par], sem)

            def wait_gather(k):
                par = k & 1
                pltpu.make_async_copy(gref.at[gidx.at[par]], gbuf.at[par],
                                      sem).wait()

            def fire_add(k):
                par = k & 1
                pltpu.make_async_copy(gbuf.at[par], ch_sh.at[sidx.at[par]],
                                      sem2).start(add=True)

            def wait_add(k):
                par = k & 1
                pltpu.make_async_copy(gbuf.at[par], ch_sh.at[sidx.at[par]],
                                      sem2).wait()

            nbatch = (nq + BATCH - 1) // BATCH

            def body(k, carry):
                @pl.when(k >= 2)
                def _():
                    wait_add(k - 2)
                build_fire(k)

                @pl.when(k >= 1)
                def _():
                    wait_gather(k - 1)
                    fire_add(k - 1)
                return carry
            lax.fori_loop(0, nbatch, body, 0)

            @pl.when(nbatch > 0)
            def _():
                wait_gather(nbatch - 1)
                fire_add(nbatch - 1)
                wait_add(nbatch - 1)

            @pl.when(nbatch > 1)
            def _():
                wait_add(nbatch - 2)

            plsc.subcore_barrier()

            @pl.when(sid < 15)
            def _():
                pltpu.sync_copy(ch_sh.at[pl.ds(392 * sid, 392)],
                                href.at[chunk, pl.ds(392 * sid, 392)])

            @pl.when(sid == 15)
            def _():
                pltpu.sync_copy(ch_sh.at[pl.ds(392 * 15, 376)],
                                href.at[chunk, pl.ds(392 * 15, 376)])

            plsc.subcore_barrier()


def _scale_body(deg_ref, f_ref, o_ref):
    w = 1.0 / (jnp.sqrt(deg_ref[...]) + 1e-8)
    o_ref[...] = f_ref[...] * w


def _scale(deg3, feats, rows, nb):
    blk = rows // nb
    return pl.pallas_call(
        _scale_body,
        grid=(G, nb),
        in_specs=[
            pl.BlockSpec((1, blk, 1), lambda b, i: (b, i, 0)),
            pl.BlockSpec((1, blk, D), lambda b, i: (b, i, 0)),
        ],
        out_specs=pl.BlockSpec((1, blk, D), lambda b, i: (b, i, 0)),
        out_shape=jax.ShapeDtypeStruct((G, rows, D), jnp.float32),
    )(deg3, feats)


def _post_body(deg_ref, hp_ref, acc_ref, acc_out, *rest, nparts, denom,
               want_g):
    w = 1.0 / (jnp.sqrt(deg_ref[...]) + 1e-8)
    h = hp_ref[:, 0]
    for p in range(1, nparts):
        h = h + hp_ref[:, p]
    f = (w * h) * (1.0 / denom)
    nrm = jnp.sqrt(jnp.sum(f * f, axis=2, keepdims=True))
    acc_out[...] = acc_ref[...] + f / jnp.maximum(nrm, 1e-12)
    if want_g:
        rest[0][...] = w * f


def _post(deg3, hparts, acc, rows, nparts, denom, want_g, nb):
    blk = rows // nb
    out_shape = [jax.ShapeDtypeStruct((G, rows, D), jnp.float32)]
    out_specs = [pl.BlockSpec((1, blk, D), lambda b, i: (b, i, 0))]
    if want_g:
        out_shape.append(jax.ShapeDtypeStruct((G, rows, D), jnp.float32))
        out_specs.append(pl.BlockSpec((1, blk, D), lambda b, i: (b, i, 0)))
    return pl.pallas_call(
        functools.partial(_post_body, nparts=nparts, denom=denom,
                          want_g=want_g),
        grid=(G, nb),
        in_specs=[
            pl.BlockSpec((1, blk, 1), lambda b, i: (b, i, 0)),
            pl.BlockSpec((1, nparts, blk, D), lambda b, i: (b, 0, i, 0)),
            pl.BlockSpec((1, blk, D), lambda b, i: (b, i, 0)),
        ],
        out_specs=out_specs,
        out_shape=out_shape,
    )(deg3, hparts, acc)


def kernel(edge_rows_0, edge_cols_0, edge_rows_1, edge_cols_1,
           warehouse_features, site_features):
    pad = lambda a: jnp.pad(a, (0, E_PAD - E))
    r0, c0 = pad(edge_rows_0), pad(edge_cols_0)
    r1, c1 = pad(edge_rows_1), pad(edge_cols_1)
    zeros2d = jnp.zeros((1024, D), jnp.float32)
    zeros1d = jnp.zeros((3200,), jnp.float32)
    ones128 = jnp.ones((BATCH,), jnp.float32)


    bp0, bp1, cnts, dw0, dw1, ds0, ds1 = _bin_kernel(
        r0, c0, r1, c1, ones128, zeros1d)
    degw3 = jnp.stack([dw0, dw1])[:, :, None]
    degs3 = jnp.stack([ds0, ds1])[:, :, None]

    gw = _scale(degw3, warehouse_features, W, 10)
    gs = _scale(degs3, site_features, S, 50)
    accw, accs = warehouse_features, site_features

    for i in range(2):
        hw0, hw1 = _spmm_w_kernel(r0, c0, gs[0], r1, c1, gs[1], zeros2d)
        hs0, hs1 = _spmm_s_kernel(bp0, gw[0], bp1, gw[1], cnts, zeros2d)
        hw = jnp.stack([hw0, hw1])[:, :, :W]
        hs = jnp.stack([hs0, hs1])[:, :, :CHUNK].reshape(G, 1, S, D)
        want_g = i == 0
        if want_g:
            accw, gw = _post(degw3, hw, accw, W, 2, i + 2, True, 10)
            accs, gs = _post(degs3, hs, accs, S, 1, i + 2, True, 50)
        else:
            accw, = _post(degw3, hw, accw, W, 2, i + 2, False, 10)
            accs, = _post(degs3, hs, accs, S, 1, i + 2, False, 50)
    return accw, accs
